# Initial kernel scaffold; baseline (speedup 1.0000x reference)
import jax
import jax.numpy as jnp
from jax.experimental import pallas as pl


def kernel(x, labels, w_ih_f, w_hh_f, b_ih_f, b_hh_f, w_ih_b, w_hh_b, b_ih_b, b_hh_b, W1, b1, W2, b2, crf_start, crf_end, crf_trans):
    raise NotImplementedError("write your pallas kernel here")


# speedup vs baseline: 45.9956x; 45.9956x over previous
import jax
import jax.numpy as jnp
from jax.experimental import pallas as pl
from jax.experimental.pallas import tpu as pltpu

_B, _T, _DIN, _H, _HID2, _K = 128, 512, 768, 128, 128, 13
_S = _B
_NSQ = 9


def _gates(g):
    i = jax.nn.sigmoid(g[:, 0:_H])
    f = jax.nn.sigmoid(g[:, _H:2 * _H])
    gg = jnp.tanh(g[:, 2 * _H:3 * _H])
    o = jax.nn.sigmoid(g[:, 3 * _H:4 * _H])
    return i, f, gg, o


def _lstm_body(x_ref, wih_ref, whh0_ref, whh1_ref, b_ref, hs_ref):
    pre = (jnp.dot(x_ref[...], wih_ref[...],
                   preferred_element_type=jnp.float32) + b_ref[...])
    h0 = jnp.zeros((1, _H), jnp.float32)
    c0 = jnp.zeros((1, _H), jnp.float32)
    h1 = jnp.zeros((1, _H), jnp.float32)
    c1 = jnp.zeros((1, _H), jnp.float32)
    g4 = 4 * _H
    for s in range(_S):
        r = _S - 1 - s
        g0 = pre[s:s + 1, 0:g4] + jnp.dot(
            h0, whh0_ref[...], preferred_element_type=jnp.float32)
        g1 = pre[r:r + 1, g4:2 * g4] + jnp.dot(
            h1, whh1_ref[...], preferred_element_type=jnp.float32)
        i0, f0, gg0, o0 = _gates(g0)
        i1, f1, gg1, o1 = _gates(g1)
        c0 = f0 * c0 + i0 * gg0
        h0 = o0 * jnp.tanh(c0)
        c1 = f1 * c1 + i1 * gg1
        h1 = o1 * jnp.tanh(c1)
        hs_ref[s:s + 1, 0:_H] = h0
        hs_ref[r:r + 1, _H:2 * _H] = h1


def _head_body(last_ref, w1t_ref, b1_ref, w2t_ref, b2_ref, lab_ref,
               tf0_ref, tf1_ref, cs_tab_ref, ce_tab_ref,
               csb_ref, ceb_ref, transb_ref, out_ref):
    last = last_ref[...]
    hidden = jnp.maximum(
        jnp.dot(last, w1t_ref[...], preferred_element_type=jnp.float32)
        + b1_ref[...], 0.0)
    em = (jnp.dot(hidden, w2t_ref[...], preferred_element_type=jnp.float32)
          + b2_ref[...])

    lab = lab_ref[...]

    e_tags = jnp.take_along_axis(em, lab, axis=1)
    start_v = jnp.take_along_axis(cs_tab_ref[...], lab[:, 0:1], axis=1)
    end_v = jnp.take_along_axis(ce_tab_ref[...], lab[:, _T - 1:_T], axis=1)
    l1 = lab[:, :_T - 1]
    l2 = lab[:, 1:]
    v = l1 * _K + l2
    v0 = jnp.minimum(v, 127)
    v1 = jnp.clip(v - 128, 0, _K * _K - 129)
    tv0 = jnp.take_along_axis(tf0_ref[...], v0, axis=1)
    tv1 = jnp.take_along_axis(tf1_ref[...], v1, axis=1)
    tv = jnp.where(v < 128, tv0, tv1)
    score_total = (jnp.sum(e_tags) + jnp.sum(start_v) + jnp.sum(end_v)
                   + jnp.sum(tv))

    emt = em.T
    alpha = csb_ref[...] + emt
    p = transb_ref[...] + emt[None, :, :]

    for it in range(_NSQ):
        terms = [alpha[k:k + 1, :] + p[k] for k in range(_K)]
        mx = terms[0]
        for k in range(1, _K):
            mx = jnp.maximum(mx, terms[k])
        acc = jnp.exp(terms[0] - mx)
        for k in range(1, _K):
            acc += jnp.exp(terms[k] - mx)
        alpha = mx + jnp.log(acc)
        if it < _NSQ - 1:
            mx2 = p[:, 0:1, :] + p[0][None, :, :]
            for k in range(1, _K):
                mx2 = jnp.maximum(mx2, p[:, k:k + 1, :] + p[k][None, :, :])
            acc2 = jnp.exp(p[:, 0:1, :] + p[0][None, :, :] - mx2)
            for k in range(1, _K):
                acc2 += jnp.exp(p[:, k:k + 1, :] + p[k][None, :, :] - mx2)
            p = mx2 + jnp.log(acc2)

    vz = alpha + ceb_ref[...]
    mz = jnp.max(vz, axis=0, keepdims=True)
    logz = mz + jnp.log(jnp.sum(jnp.exp(vz - mz), axis=0, keepdims=True))
    out_ref[...] = jnp.broadcast_to(jnp.sum(logz) - score_total, (1, 1))


def kernel(x, labels, w_ih_f, w_hh_f, b_ih_f, b_hh_f, w_ih_b, w_hh_b,
           b_ih_b, b_hh_b, W1, b1, W2, b2, crf_start, crf_end, crf_trans):
    f32 = jnp.float32
    xseq = x[:, -1, :]
    wih = jnp.concatenate([w_ih_f.T, w_ih_b.T], axis=1)
    bias = jnp.concatenate([b_ih_f + b_hh_f, b_ih_b + b_hh_b])[None, :]

    last = pl.pallas_call(
        _lstm_body,
        out_shape=jax.ShapeDtypeStruct((_S, 2 * _H), f32),
        name="bilstm_lastcol",
    )(xseq, wih, w_hh_f.T, w_hh_b.T, bias)

    tf = crf_trans.reshape(-1)
    tf0 = jnp.broadcast_to(tf[:128][None, :], (_B, 128))
    tf1 = jnp.broadcast_to(
        jnp.pad(tf[128:], (0, 128 - (_K * _K - 128)))[None, :], (_B, 128))
    cs_tab = jnp.broadcast_to(crf_start[None, :], (_B, _K))
    ce_tab = jnp.broadcast_to(crf_end[None, :], (_B, _K))
    csb = jnp.broadcast_to(crf_start[:, None], (_K, _B))
    ceb = jnp.broadcast_to(crf_end[:, None], (_K, _B))
    transb = jnp.broadcast_to(crf_trans[:, :, None], (_K, _K, _B))

    out = pl.pallas_call(
        _head_body,
        out_shape=jax.ShapeDtypeStruct((1, 1), f32),
        name="mlp_crf_nll",
    )(last, W1.T, b1[None, :], W2.T, b2[None, :], labels,
      tf0, tf1, cs_tab, ce_tab, csb, ceb, transb)
    return out[0, 0]


# speedup vs baseline: 73.0709x; 1.5887x over previous
import jax
import jax.numpy as jnp
from jax import lax
from jax.experimental import pallas as pl
from jax.experimental.pallas import tpu as pltpu

_B, _T, _DIN, _H, _HID2, _K = 128, 512, 768, 128, 128, 13
_S = _B
_NSQ = 9

_TRANS_B = (((1,), (1,)), ((), ()))


def _gates(g):
    i = jax.nn.sigmoid(g[:, 0:_H])
    f = jax.nn.sigmoid(g[:, _H:2 * _H])
    gg = jnp.tanh(g[:, 2 * _H:3 * _H])
    o = jax.nn.sigmoid(g[:, 3 * _H:4 * _H])
    return i, f, gg, o


def _dot_t(a, b):
    return lax.dot_general(a, b, _TRANS_B, preferred_element_type=jnp.float32)


def _body(x_ref, lab_ref, wihf_ref, whhf_ref, wihb_ref, whhb_ref,
          bif_ref, bhf_ref, bib_ref, bhb_ref, w1_ref, b1_ref, w2_ref,
          b2_ref, cs_ref, ce_ref, trans_ref, tfrow_ref, out_ref, hs_ref):
    xcol = x_ref[:, 7, :]
    pre0 = _dot_t(xcol, wihf_ref[...]) + bif_ref[...] + bhf_ref[...]
    pre1 = _dot_t(xcol, wihb_ref[...]) + bib_ref[...] + bhb_ref[...]
    whh0 = jnp.transpose(whhf_ref[...])
    whh1 = jnp.transpose(whhb_ref[...])

    h0 = jnp.zeros((1, _H), jnp.float32)
    c0 = jnp.zeros((1, _H), jnp.float32)
    h1 = jnp.zeros((1, _H), jnp.float32)
    c1 = jnp.zeros((1, _H), jnp.float32)
    for s in range(_S):
        r = _S - 1 - s
        g0 = pre0[s:s + 1, :] + jnp.dot(h0, whh0,
                                        preferred_element_type=jnp.float32)
        g1 = pre1[r:r + 1, :] + jnp.dot(h1, whh1,
                                        preferred_element_type=jnp.float32)
        i0, f0, gg0, o0 = _gates(g0)
        i1, f1, gg1, o1 = _gates(g1)
        c0 = f0 * c0 + i0 * gg0
        h0 = o0 * jnp.tanh(c0)
        c1 = f1 * c1 + i1 * gg1
        h1 = o1 * jnp.tanh(c1)
        hs_ref[s:s + 1, 0:_H] = h0
        hs_ref[r:r + 1, _H:2 * _H] = h1

    last = hs_ref[...]
    hidden = jnp.maximum(_dot_t(last, w1_ref[...]) + b1_ref[...], 0.0)
    em = _dot_t(hidden, w2_ref[...]) + b2_ref[...]

    lab = lab_ref[...]

    e_tags = jnp.take_along_axis(em, lab, axis=1)
    cs_tab = jnp.broadcast_to(cs_ref[...], (_B, _K))
    ce_tab = jnp.broadcast_to(ce_ref[...], (_B, _K))
    start_v = jnp.take_along_axis(cs_tab, lab[:, 0:1], axis=1)
    end_v = jnp.take_along_axis(ce_tab, lab[:, _T - 1:_T], axis=1)
    l1 = lab[:, :_T - 1]
    l2 = lab[:, 1:]
    v = l1 * _K + l2
    v0 = jnp.minimum(v, 127)
    v1 = jnp.clip(v - 128, 0, _K * _K - 129)
    tf0 = jnp.broadcast_to(tfrow_ref[:, 0:128], (_B, 128))
    tf1 = jnp.broadcast_to(tfrow_ref[:, 128:256], (_B, 128))
    tv0 = jnp.take_along_axis(tf0, v0, axis=1)
    tv1 = jnp.take_along_axis(tf1, v1, axis=1)
    tv = jnp.where(v < 128, tv0, tv1)
    score_total = (jnp.sum(e_tags) + jnp.sum(start_v) + jnp.sum(end_v)
                   + jnp.sum(tv))

    emt = em.T
    alpha = jnp.transpose(cs_ref[...]) + emt
    p = trans_ref[...][:, :, None] + emt[None, :, :]

    for it in range(_NSQ):
        terms = [alpha[k:k + 1, :] + p[k] for k in range(_K)]
        mx = terms[0]
        for k in range(1, _K):
            mx = jnp.maximum(mx, terms[k])
        acc = jnp.exp(terms[0] - mx)
        for k in range(1, _K):
            acc += jnp.exp(terms[k] - mx)
        alpha = mx + jnp.log(acc)
        if it < _NSQ - 1:
            mx2 = p[:, 0:1, :] + p[0][None, :, :]
            for k in range(1, _K):
                mx2 = jnp.maximum(mx2, p[:, k:k + 1, :] + p[k][None, :, :])
            acc2 = jnp.exp(p[:, 0:1, :] + p[0][None, :, :] - mx2)
            for k in range(1, _K):
                acc2 += jnp.exp(p[:, k:k + 1, :] + p[k][None, :, :] - mx2)
            p = mx2 + jnp.log(acc2)

    vz = alpha + jnp.transpose(ce_ref[...])
    mz = jnp.max(vz, axis=0, keepdims=True)
    logz = mz + jnp.log(jnp.sum(jnp.exp(vz - mz), axis=0, keepdims=True))
    out_ref[...] = jnp.broadcast_to(jnp.sum(logz) - score_total, (1, 1))


def kernel(x, labels, w_ih_f, w_hh_f, b_ih_f, b_hh_f, w_ih_b, w_hh_b,
           b_ih_b, b_hh_b, W1, b1, W2, b2, crf_start, crf_end, crf_trans):
    f32 = jnp.float32
    tfrow = jnp.pad(crf_trans.reshape(1, -1), ((0, 0), (0, 256 - _K * _K)))

    full = pl.BlockSpec
    out = pl.pallas_call(
        _body,
        in_specs=[
            pl.BlockSpec((_S, 8, _DIN), lambda i: (0, _T // 8 - 1, 0)),
            full((_B, _T), lambda i: (0, 0)),
            full((4 * _H, _DIN), lambda i: (0, 0)),
            full((4 * _H, _H), lambda i: (0, 0)),
            full((4 * _H, _DIN), lambda i: (0, 0)),
            full((4 * _H, _H), lambda i: (0, 0)),
            full((1, 4 * _H), lambda i: (0, 0)),
            full((1, 4 * _H), lambda i: (0, 0)),
            full((1, 4 * _H), lambda i: (0, 0)),
            full((1, 4 * _H), lambda i: (0, 0)),
            full((_HID2, 2 * _H), lambda i: (0, 0)),
            full((1, _HID2), lambda i: (0, 0)),
            full((_K, _HID2), lambda i: (0, 0)),
            full((1, _K), lambda i: (0, 0)),
            full((1, _K), lambda i: (0, 0)),
            full((1, _K), lambda i: (0, 0)),
            full((_K, _K), lambda i: (0, 0)),
            full((1, 256), lambda i: (0, 0)),
        ],
        out_specs=pl.BlockSpec((1, 1), lambda i: (0, 0)),
        grid=(1,),
        out_shape=jax.ShapeDtypeStruct((1, 1), f32),
        scratch_shapes=[pltpu.VMEM((_S, 2 * _H), f32)],
        name="bilstm_mlp_crf_nll",
    )(x, labels, w_ih_f, w_hh_f, w_ih_b, w_hh_b,
      b_ih_f[None, :], b_hh_f[None, :], b_ih_b[None, :], b_hh_b[None, :],
      W1, b1[None, :], W2, b2[None, :],
      crf_start[None, :], crf_end[None, :], crf_trans, tfrow)
    return out[0, 0]
